# Initial kernel scaffold; baseline (speedup 1.0000x reference)
import jax
import jax.numpy as jnp
from jax.experimental import pallas as pl


def kernel(users_emb, items_emb, user_indices, user_values, item_indices, item_values):
    raise NotImplementedError("write your pallas kernel here")


# speedup vs baseline: 2.4997x; 2.4997x over previous
import jax
import jax.numpy as jnp
from jax import lax
from jax.experimental import pallas as pl
from jax.experimental.pallas import tpu as pltpu
from jax.experimental.pallas import tpu_sc as plsc

N_ROWS = 10000
D = 128
HALF_D = 64
E = 320000

NUM_CORES = 2
NUM_SUBCORES = 16
CHUNK = 128
NCH = -(-E // (NUM_SUBCORES * CHUNK))
E_PAD = NUM_SUBCORES * CHUNK * NCH
ROWS_PER_TILE = N_ROWS // NUM_SUBCORES


def _spmm_body(emb_hbm, cols_hbm, rows_hbm, vals_hbm, zeros_hbm, out_hbm,
               colv, roww, valv, gbuf, accum, sem):
    c = lax.axis_index("c")
    s = lax.axis_index("s")

    pltpu.sync_copy(rows_hbm.at[c, s], roww)
    pltpu.sync_copy(vals_hbm.at[c, s], valv)

    for p in range(2):
        pltpu.sync_copy(zeros_hbm, accum.at[pl.ds(s * ROWS_PER_TILE, ROWS_PER_TILE)])
        pltpu.sync_copy(cols_hbm.at[p, c, s], colv)
        plsc.subcore_barrier()

        def chunk_body(j, carry):
            pltpu.async_copy(emb_hbm.at[colv.at[j]], gbuf, sem).wait()

            def group_body(g, carry2):
                vvec = valv[j, pl.ds(g * 16, 16)]
                for lane in range(16):
                    vv = jnp.full((16,), vvec[lane], dtype=jnp.float32)
                    e = g * 16 + lane
                    for d in range(HALF_D // 16):
                        sl = pl.ds(d * 16, 16)
                        gbuf[e, sl] = gbuf[e, sl] * vv
                return carry2

            lax.fori_loop(0, CHUNK // 16, group_body, 0, unroll=False)

            pltpu.sync_copy(gbuf, accum.at[roww.at[j]], add=True)
            return carry

        lax.fori_loop(0, NCH, chunk_body, 0, unroll=False)

        plsc.subcore_barrier()

        pltpu.sync_copy(
            accum.at[pl.ds(s * ROWS_PER_TILE, ROWS_PER_TILE)],
            out_hbm.at[c, p, s],
        )


@jax.jit
def kernel(users_emb, items_emb, user_indices, user_values, item_indices, item_values):
    emb = jnp.concatenate([users_emb, items_emb], axis=0)
    emb = emb.reshape(2 * N_ROWS, 2, HALF_D).transpose(1, 0, 2)
    emb = emb.reshape(4 * N_ROWS, HALF_D)

    def prep_idx(a):
        a = a.astype(jnp.int32)
        a = jnp.concatenate([a, jnp.zeros((E_PAD - E,), jnp.int32)])
        return a.reshape(NUM_SUBCORES, NCH, CHUNK)

    base_cols = jnp.stack([
        prep_idx(user_indices[1]),
        prep_idx(item_indices[1] + N_ROWS),
    ])
    cols = jnp.stack([base_cols, base_cols + 2 * N_ROWS])
    rows = jnp.stack([
        prep_idx(user_indices[0]),
        prep_idx(item_indices[0]),
    ])

    def prep_val(v):
        v = jnp.concatenate([v, jnp.zeros((E_PAD - E,), jnp.float32)])
        return v.reshape(NUM_SUBCORES, NCH, CHUNK)

    vals = jnp.stack([prep_val(user_values), prep_val(item_values)])
    zeros = jnp.zeros((ROWS_PER_TILE, HALF_D), jnp.float32)

    mesh = plsc.VectorSubcoreMesh(
        core_axis_name="c", subcore_axis_name="s",
        num_cores=NUM_CORES, num_subcores=NUM_SUBCORES,
    )
    out = pl.kernel(
        _spmm_body,
        out_type=jax.ShapeDtypeStruct(
            (NUM_CORES, 2, NUM_SUBCORES, ROWS_PER_TILE, HALF_D), jnp.float32),
        mesh=mesh,
        compiler_params=pltpu.CompilerParams(use_tc_tiling_on_sc=False),
        scratch_types=[
            pltpu.VMEM((NCH, CHUNK), jnp.int32),
            pltpu.VMEM((NCH, CHUNK), jnp.int32),
            pltpu.VMEM((NCH, CHUNK), jnp.float32),
            pltpu.VMEM((CHUNK, HALF_D), jnp.float32),
            pltpu.VMEM_SHARED((N_ROWS, HALF_D), jnp.float32),
            pltpu.SemaphoreType.DMA,
        ],
    )(emb, cols, rows, vals, zeros)

    out = out.transpose(0, 2, 3, 1, 4).reshape(NUM_CORES, N_ROWS, D)
    return (out[0], out[1])


# speedup vs baseline: 2.8549x; 1.1421x over previous
import jax
import jax.numpy as jnp
from jax import lax
from jax.experimental import pallas as pl
from jax.experimental.pallas import tpu as pltpu
from jax.experimental.pallas import tpu_sc as plsc

N_ROWS = 10000
D = 128
HALF_D = 64
E = 320000

NUM_CORES = 2
NUM_SUBCORES = 16
CHUNK = 128
NBUF = 2
NCH = 158
E_PAD = NUM_SUBCORES * CHUNK * NCH
ROWS_PER_TILE = N_ROWS // NUM_SUBCORES


def _spmm_body(emb_hbm, cols_hbm, rows_hbm, vals_hbm, zeros_hbm, out_hbm,
               colv, roww, valv, gbuf0, gbuf1, accum, sem0, sem1):
    c = lax.axis_index("c")
    s = lax.axis_index("s")
    gbufs = (gbuf0, gbuf1)
    sems = (sem0, sem1)

    pltpu.sync_copy(rows_hbm.at[c, s], roww)
    pltpu.sync_copy(vals_hbm.at[c, s], valv)

    for p in range(2):
        pltpu.sync_copy(zeros_hbm, accum.at[pl.ds(s * ROWS_PER_TILE, ROWS_PER_TILE)])
        pltpu.sync_copy(cols_hbm.at[p, c, s], colv)
        plsc.subcore_barrier()

        for b in range(NBUF):
            pltpu.async_copy(emb_hbm.at[colv.at[b]], gbufs[b], sems[b])

        def pair_body(jj, carry):
            for b in range(NBUF):
                j = jj * NBUF + b
                pltpu.make_async_copy(emb_hbm.at[colv.at[j]], gbufs[b], sems[b]).wait()

                def group_body(g, carry2, _b=b, _j=j):
                    vvec = valv[_j, pl.ds(g * 16, 16)]
                    for lane in range(16):
                        vv = jnp.full((16,), vvec[lane], dtype=jnp.float32)
                        e = g * 16 + lane
                        for d in range(HALF_D // 16):
                            sl = pl.ds(d * 16, 16)
                            gbufs[_b][e, sl] = gbufs[_b][e, sl] * vv
                    return carry2

                lax.fori_loop(0, CHUNK // 16, group_body, 0, unroll=False)

                pltpu.sync_copy(gbufs[b], accum.at[roww.at[j]], add=True)

                @pl.when(jj + 1 < NCH // NBUF)
                def _(_b=b, _j=j):
                    pltpu.async_copy(
                        emb_hbm.at[colv.at[_j + NBUF]], gbufs[_b], sems[_b])
            return carry

        lax.fori_loop(0, NCH // NBUF, pair_body, 0, unroll=False)

        plsc.subcore_barrier()

        pltpu.sync_copy(
            accum.at[pl.ds(s * ROWS_PER_TILE, ROWS_PER_TILE)],
            out_hbm.at[c, p, s],
        )


@jax.jit
def kernel(users_emb, items_emb, user_indices, user_values, item_indices, item_values):
    emb = jnp.concatenate([users_emb, items_emb], axis=0)
    emb = emb.reshape(2 * N_ROWS, 2, HALF_D).transpose(1, 0, 2)
    emb = emb.reshape(4 * N_ROWS, HALF_D)

    def prep_idx(a):
        a = a.astype(jnp.int32)
        a = jnp.concatenate([a, jnp.zeros((E_PAD - E,), jnp.int32)])
        return a.reshape(NUM_SUBCORES, NCH, CHUNK)

    base_cols = jnp.stack([
        prep_idx(user_indices[1]),
        prep_idx(item_indices[1] + N_ROWS),
    ])
    cols = jnp.stack([base_cols, base_cols + 2 * N_ROWS])
    rows = jnp.stack([
        prep_idx(user_indices[0]),
        prep_idx(item_indices[0]),
    ])

    def prep_val(v):
        v = jnp.concatenate([v, jnp.zeros((E_PAD - E,), jnp.float32)])
        return v.reshape(NUM_SUBCORES, NCH, CHUNK)

    vals = jnp.stack([prep_val(user_values), prep_val(item_values)])
    zeros = jnp.zeros((ROWS_PER_TILE, HALF_D), jnp.float32)

    mesh = plsc.VectorSubcoreMesh(
        core_axis_name="c", subcore_axis_name="s",
        num_cores=NUM_CORES, num_subcores=NUM_SUBCORES,
    )
    out = pl.kernel(
        _spmm_body,
        out_type=jax.ShapeDtypeStruct(
            (NUM_CORES, 2, NUM_SUBCORES, ROWS_PER_TILE, HALF_D), jnp.float32),
        mesh=mesh,
        compiler_params=pltpu.CompilerParams(use_tc_tiling_on_sc=False),
        scratch_types=[
            pltpu.VMEM((NCH, CHUNK), jnp.int32),
            pltpu.VMEM((NCH, CHUNK), jnp.int32),
            pltpu.VMEM((NCH, CHUNK), jnp.float32),
            pltpu.VMEM((CHUNK, HALF_D), jnp.float32),
            pltpu.VMEM((CHUNK, HALF_D), jnp.float32),
            pltpu.VMEM_SHARED((N_ROWS, HALF_D), jnp.float32),
            pltpu.SemaphoreType.DMA,
            pltpu.SemaphoreType.DMA,
        ],
    )(emb, cols, rows, vals, zeros)

    out = out.transpose(0, 2, 3, 1, 4).reshape(NUM_CORES, N_ROWS, D)
    return (out[0], out[1])


# speedup vs baseline: 5.0021x; 1.7521x over previous
import jax
import jax.numpy as jnp
from jax import lax
from jax.experimental import pallas as pl
from jax.experimental.pallas import tpu as pltpu
from jax.experimental.pallas import tpu_sc as plsc

N_ROWS = 10000
D = 128
E = 320000

NUM_CORES = 2
NUM_SUBCORES = 16
CHUNK = 128
NCH = 158
E_PAD = NUM_SUBCORES * CHUNK * NCH
ROWS_PER_TILE = N_ROWS // NUM_SUBCORES
COL, ROW, VAL = 0, 1, 2


def _spmm_body(emb_hbm, recs_hbm, zeros_hbm, out_hbm,
               ibuf0, ibuf1, gbuf0, gbuf1, sbuf, accum,
               isem0, isem1, gsem0, gsem1):
    c = lax.axis_index("c")
    s = lax.axis_index("s")
    ibufs = (ibuf0, ibuf1)
    gbufs = (gbuf0, gbuf1)
    isems = (isem0, isem1)
    gsems = (gsem0, gsem1)

    pltpu.sync_copy(zeros_hbm, accum.at[pl.ds(s * ROWS_PER_TILE, ROWS_PER_TILE)])

    for b in range(2):
        pltpu.async_copy(recs_hbm.at[c, s, b], ibufs[b], isems[b])
    plsc.subcore_barrier()
    pltpu.make_async_copy(recs_hbm.at[c, s, 0], ibuf0, isem0).wait()
    pltpu.async_copy(emb_hbm.at[ibuf0.at[COL]], gbuf0, gsem0)

    npair = NCH // 2

    def pair_body(jj, carry):
        for b in range(2):
            j = jj * 2 + b
            o = 1 - b

            @pl.when(jj * 2 + b + 1 < NCH)
            def _(_b=b, _o=o, _j=j):
                pltpu.make_async_copy(
                    recs_hbm.at[c, s, 0], ibufs[_o], isems[_o]).wait()
                pltpu.async_copy(
                    emb_hbm.at[ibufs[_o].at[COL]], gbufs[_o], gsems[_o])

            pltpu.make_async_copy(
                emb_hbm.at[ibufs[b].at[COL]], gbufs[b], gsems[b]).wait()

            @plsc.parallel_loop(0, CHUNK // 16, unroll=2)
            def group_body(g, _b=b):
                vvec = lax.bitcast_convert_type(
                    ibufs[_b][VAL, pl.ds(g * 16, 16)], jnp.float32)
                for lane in range(16):
                    vv = jnp.full((16,), vvec[lane], dtype=jnp.float32)
                    e = g * 16 + lane
                    for d in range(D // 16):
                        sl = pl.ds(d * 16, 16)
                        sbuf[e, sl] = gbufs[_b][e, sl] * vv

            pltpu.sync_copy(sbuf, accum.at[ibufs[b].at[ROW]], add=True)

            @pl.when(jj * 2 + b + 2 < NCH)
            def _(_b=b, _j=j):
                pltpu.async_copy(recs_hbm.at[c, s, _j + 2], ibufs[_b], isems[_b])
        return carry

    lax.fori_loop(0, npair, pair_body, 0, unroll=False)

    plsc.subcore_barrier()

    pltpu.sync_copy(
        accum.at[pl.ds(s * ROWS_PER_TILE, ROWS_PER_TILE)],
        out_hbm.at[c, s],
    )


@jax.jit
def kernel(users_emb, items_emb, user_indices, user_values, item_indices, item_values):
    emb = jnp.concatenate([users_emb, items_emb], axis=0)

    def prep(a):
        a = a.astype(jnp.int32)
        a = jnp.concatenate([a, jnp.zeros((E_PAD - E,), jnp.int32)])
        return a.reshape(NUM_SUBCORES, NCH, 1, CHUNK)

    recs = jnp.stack([
        jnp.concatenate([
            prep(user_indices[1]),
            prep(user_indices[0]),
            prep(lax.bitcast_convert_type(user_values, jnp.int32)),
        ], axis=2),
        jnp.concatenate([
            prep(item_indices[1] + N_ROWS),
            prep(item_indices[0]),
            prep(lax.bitcast_convert_type(item_values, jnp.int32)),
        ], axis=2),
    ])
    zeros = jnp.zeros((ROWS_PER_TILE, D), jnp.float32)

    mesh = plsc.VectorSubcoreMesh(
        core_axis_name="c", subcore_axis_name="s",
        num_cores=NUM_CORES, num_subcores=NUM_SUBCORES,
    )
    out = pl.kernel(
        _spmm_body,
        out_type=jax.ShapeDtypeStruct(
            (NUM_CORES, NUM_SUBCORES, ROWS_PER_TILE, D), jnp.float32),
        mesh=mesh,
        compiler_params=pltpu.CompilerParams(use_tc_tiling_on_sc=False),
        scratch_types=[
            pltpu.VMEM((3, CHUNK), jnp.int32),
            pltpu.VMEM((3, CHUNK), jnp.int32),
            pltpu.VMEM((CHUNK, D), jnp.float32),
            pltpu.VMEM((CHUNK, D), jnp.float32),
            pltpu.VMEM((CHUNK, D), jnp.float32),
            pltpu.VMEM_SHARED((N_ROWS, D), jnp.float32),
            pltpu.SemaphoreType.DMA,
            pltpu.SemaphoreType.DMA,
            pltpu.SemaphoreType.DMA,
            pltpu.SemaphoreType.DMA,
        ],
    )(emb, recs, zeros)

    out = out.reshape(NUM_CORES, N_ROWS, D)
    return (out[0], out[1])


# speedup vs baseline: 5.1392x; 1.0274x over previous
import jax
import jax.numpy as jnp
from jax import lax
from jax.experimental import pallas as pl
from jax.experimental.pallas import tpu as pltpu
from jax.experimental.pallas import tpu_sc as plsc

N_ROWS = 10000
D = 128
E = 320000

NUM_CORES = 2
NUM_SUBCORES = 16
CHUNK = 128
NCH = 158
E_PAD = NUM_SUBCORES * CHUNK * NCH
ROWS_PER_TILE = N_ROWS // NUM_SUBCORES
COL, ROW, VAL = 0, 1, 2


def _spmm_body(emb_hbm, recs_hbm, zeros_hbm, out_hbm,
               ibuf0, ibuf1, gbuf0, gbuf1, sbuf, accum,
               isem0, isem1, gsem0, gsem1):
    c = lax.axis_index("c")
    s = lax.axis_index("s")
    ibufs = (ibuf0, ibuf1)
    gbufs = (gbuf0, gbuf1)
    isems = (isem0, isem1)
    gsems = (gsem0, gsem1)

    pltpu.sync_copy(zeros_hbm, accum.at[pl.ds(s * ROWS_PER_TILE, ROWS_PER_TILE)])

    for b in range(2):
        pltpu.async_copy(recs_hbm.at[c, s, b], ibufs[b], isems[b])
    plsc.subcore_barrier()
    pltpu.make_async_copy(recs_hbm.at[c, s, 0], ibuf0, isem0).wait()
    pltpu.async_copy(emb_hbm.at[ibuf0.at[COL]], gbuf0, gsem0)

    npair = NCH // 2

    def pair_body(jj, carry):
        for b in range(2):
            j = jj * 2 + b
            o = 1 - b

            @pl.when(jj * 2 + b + 1 < NCH)
            def _(_b=b, _o=o, _j=j):
                pltpu.make_async_copy(
                    recs_hbm.at[c, s, 0], ibufs[_o], isems[_o]).wait()
                pltpu.async_copy(
                    emb_hbm.at[ibufs[_o].at[COL]], gbufs[_o], gsems[_o])

            pltpu.make_async_copy(
                emb_hbm.at[ibufs[b].at[COL]], gbufs[b], gsems[b]).wait()

            @plsc.parallel_loop(0, CHUNK // 16, unroll=1)
            def group_body(g, _b=b):
                vvec = lax.bitcast_convert_type(
                    ibufs[_b][VAL, pl.ds(g * 16, 16)], jnp.float32)
                for lane in range(16):
                    vv = jnp.full((16,), vvec[lane], dtype=jnp.float32)
                    e = g * 16 + lane
                    for d in range(D // 16):
                        sl = pl.ds(d * 16, 16)
                        sbuf[e, sl] = gbufs[_b][e, sl] * vv

            pltpu.sync_copy(sbuf, accum.at[ibufs[b].at[ROW]], add=True)

            @pl.when(jj * 2 + b + 2 < NCH)
            def _(_b=b, _j=j):
                pltpu.async_copy(recs_hbm.at[c, s, _j + 2], ibufs[_b], isems[_b])
        return carry

    lax.fori_loop(0, npair, pair_body, 0, unroll=False)

    plsc.subcore_barrier()

    pltpu.sync_copy(
        accum.at[pl.ds(s * ROWS_PER_TILE, ROWS_PER_TILE)],
        out_hbm.at[c, s],
    )


@jax.jit
def kernel(users_emb, items_emb, user_indices, user_values, item_indices, item_values):
    emb = jnp.concatenate([users_emb, items_emb], axis=0)

    def prep(a):
        a = a.astype(jnp.int32)
        a = jnp.concatenate([a, jnp.zeros((E_PAD - E,), jnp.int32)])
        return a.reshape(NUM_SUBCORES, NCH, 1, CHUNK)

    recs = jnp.stack([
        jnp.concatenate([
            prep(user_indices[1]),
            prep(user_indices[0]),
            prep(lax.bitcast_convert_type(user_values, jnp.int32)),
        ], axis=2),
        jnp.concatenate([
            prep(item_indices[1] + N_ROWS),
            prep(item_indices[0]),
            prep(lax.bitcast_convert_type(item_values, jnp.int32)),
        ], axis=2),
    ])
    zeros = jnp.zeros((ROWS_PER_TILE, D), jnp.float32)

    mesh = plsc.VectorSubcoreMesh(
        core_axis_name="c", subcore_axis_name="s",
        num_cores=NUM_CORES, num_subcores=NUM_SUBCORES,
    )
    out = pl.kernel(
        _spmm_body,
        out_type=jax.ShapeDtypeStruct(
            (NUM_CORES, NUM_SUBCORES, ROWS_PER_TILE, D), jnp.float32),
        mesh=mesh,
        compiler_params=pltpu.CompilerParams(use_tc_tiling_on_sc=False),
        scratch_types=[
            pltpu.VMEM((3, CHUNK), jnp.int32),
            pltpu.VMEM((3, CHUNK), jnp.int32),
            pltpu.VMEM((CHUNK, D), jnp.float32),
            pltpu.VMEM((CHUNK, D), jnp.float32),
            pltpu.VMEM((CHUNK, D), jnp.float32),
            pltpu.VMEM_SHARED((N_ROWS, D), jnp.float32),
            pltpu.SemaphoreType.DMA,
            pltpu.SemaphoreType.DMA,
            pltpu.SemaphoreType.DMA,
            pltpu.SemaphoreType.DMA,
        ],
    )(emb, recs, zeros)

    out = out.reshape(NUM_CORES, N_ROWS, D)
    return (out[0], out[1])
